# baseline (speedup 1.0000x reference)
import functools

import jax
import jax.numpy as jnp
from jax import lax
from jax.experimental import pallas as pl
from jax.experimental.pallas import tpu as pltpu
from jax.experimental.pallas import tpu_sc as plsc

EPS = float(jnp.finfo(jnp.float32).eps)
TRANSL_BIAS = 0.01
STEP = 6

NC, NS, L = 2, 16, 16
NW = NC * NS


def _se3_table_body(rot_ref, tra_ref, out_ref):
    F = rot_ref.shape[1]
    FP = out_ref.shape[0] // 72
    col = lax.broadcasted_iota(jnp.int32, (1, F), 1)
    mask = jnp.where((col >= 1) & (col < F - 1), 1.0, 0.0).astype(jnp.float32)
    for j in range(STEP):
        w0 = rot_ref[3 * j : 3 * j + 1, :] * mask + EPS
        w1 = rot_ref[3 * j + 1 : 3 * j + 2, :] * mask + EPS
        w2 = rot_ref[3 * j + 2 : 3 * j + 3, :] * mask + EPS
        u0 = tra_ref[3 * j : 3 * j + 1, :] * mask * TRANSL_BIAS
        u1 = tra_ref[3 * j + 1 : 3 * j + 2, :] * mask * TRANSL_BIAS
        u2 = tra_ref[3 * j + 2 : 3 * j + 3, :] * mask * TRANSL_BIAS
        sq0, sq1, sq2 = w0 * w0, w1 * w1, w2 * w2
        w01, w02, w12 = w0 * w1, w0 * w2, w1 * w2
        theta2 = sq0 + sq1 + sq2
        theta = jnp.sqrt(theta2)
        small = theta < 1e-6
        th = jnp.where(small, 1.0, theta)
        sin_t = jnp.sin(th)
        cos_t = jnp.cos(th)
        t2 = theta * theta
        A = jnp.where(small, 1.0 - t2 / 6.0, sin_t / th)
        B = jnp.where(small, 0.5 - t2 / 24.0, (1.0 - cos_t) / (th * th))
        C = jnp.where(small, 1.0 / 6.0 - t2 / 120.0, (th - sin_t) / (th * th * th))
        r00 = 1.0 - B * (sq1 + sq2)
        r01 = -A * w2 + B * w01
        r02 = A * w1 + B * w02
        r10 = A * w2 + B * w01
        r11 = 1.0 - B * (sq0 + sq2)
        r12 = -A * w0 + B * w12
        r20 = -A * w1 + B * w02
        r21 = A * w0 + B * w12
        r22 = 1.0 - B * (sq0 + sq1)
        v00 = 1.0 - C * (sq1 + sq2)
        v01 = -B * w2 + C * w01
        v02 = B * w1 + C * w02
        v10 = B * w2 + C * w01
        v11 = 1.0 - C * (sq0 + sq2)
        v12 = -B * w0 + C * w12
        v20 = -B * w1 + C * w02
        v21 = B * w0 + C * w12
        v22 = 1.0 - C * (sq0 + sq1)
        t0 = v00 * u0 + v01 * u1 + v02 * u2
        t1 = v10 * u0 + v11 * u1 + v12 * u2
        t2_ = v20 * u0 + v21 * u1 + v22 * u2
        rows = [r00, r01, r02, r10, r11, r12, r20, r21, r22, t0, t1, t2_]
        for d, v in enumerate(rows):
            out_ref[pl.ds((j * 12 + d) * FP, F)] = jnp.reshape(v, (F,))


@functools.lru_cache(maxsize=None)
def _build_compose(N, F):
    FP = ((F + 127) // 128) * 128
    RPW = N // NW
    CHUNK = 256
    NCHUNK = RPW // CHUNK
    GROUPS = CHUNK // L

    mesh = plsc.VectorSubcoreMesh(core_axis_name="c", subcore_axis_name="s")

    def cst(v):
        return jnp.full((L,), v, jnp.int32)

    @functools.partial(
        pl.kernel,
        mesh=mesh,
        out_type=[
            jax.ShapeDtypeStruct((9 * N * STEP,), jnp.float32),
            jax.ShapeDtypeStruct((3 * N * STEP,), jnp.float32),
        ],
        scratch_types=[
            pltpu.VMEM((72 * FP,), jnp.float32),
            [pltpu.VMEM((CHUNK,), jnp.int32) for _ in range(2)],
            [pltpu.VMEM((CHUNK * 12,), jnp.float32) for _ in range(2)],
            [pltpu.VMEM((CHUNK * 54,), jnp.float32) for _ in range(2)],
            [pltpu.VMEM((CHUNK * 18,), jnp.float32) for _ in range(2)],
            [pltpu.SemaphoreType.DMA for _ in range(2)],
            [pltpu.SemaphoreType.DMA for _ in range(2)],
        ],
        compiler_params=pltpu.CompilerParams(needs_layout_passes=False),
    )
    def compose(
        w_hbm, idx_hbm, der_hbm, r_hbm, t_hbm,
        tab_v, idx_vs, w_vs, r_vs, t_vs, in_sems, out_sems,
    ):
        wid = lax.axis_index("s") * NC + lax.axis_index("c")
        iota = lax.iota(jnp.int32, L)
        M = N * STEP
        C6 = CHUNK * 6

        def fire_in(c, b):
            base = wid * RPW + c * CHUNK
            pltpu.make_async_copy(
                idx_hbm.at[pl.ds(base, CHUNK)], idx_vs[b], in_sems[b]
            ).start()
            for d in range(12):
                pltpu.make_async_copy(
                    w_hbm.at[pl.ds(d * N + base, CHUNK)],
                    w_vs[b].at[pl.ds(d * CHUNK, CHUNK)],
                    in_sems[b],
                ).start()

        def wait_in(b):
            pltpu.make_async_copy(
                idx_hbm.at[pl.ds(0, CHUNK)], idx_vs[b], in_sems[b]
            ).wait()
            pltpu.make_async_copy(
                w_hbm.at[pl.ds(0, CHUNK * 12)], w_vs[b], in_sems[b]
            ).wait()

        def fire_out(c, b):
            base6 = (wid * RPW + c * CHUNK) * 6
            for d in range(9):
                pltpu.make_async_copy(
                    r_vs[b].at[pl.ds(d * C6, C6)],
                    r_hbm.at[pl.ds(d * M + base6, C6)],
                    out_sems[b],
                ).start()
            for d in range(3):
                pltpu.make_async_copy(
                    t_vs[b].at[pl.ds(d * C6, C6)],
                    t_hbm.at[pl.ds(d * M + base6, C6)],
                    out_sems[b],
                ).start()

        def wait_out(b):
            pltpu.make_async_copy(
                r_hbm.at[pl.ds(0, CHUNK * 54)], r_vs[b], out_sems[b]
            ).wait()
            pltpu.make_async_copy(
                t_hbm.at[pl.ds(0, CHUNK * 18)], t_vs[b], out_sems[b]
            ).wait()

        def compute(b):
            idx_v, w_v, r_v, t_v = idx_vs[b], w_vs[b], r_vs[b], t_vs[b]

            @plsc.parallel_loop(0, GROUPS, unroll=4)
            def group_body(g):
                row = g * L + iota
                t16 = idx_v[pl.ds(g * L, L)]
                row6 = row * 6
                rb = [
                    [w_v[pl.ds((r * 4 + k) * CHUNK + g * L, L)] for k in range(3)]
                    for r in range(3)
                ]
                tb = [w_v[pl.ds((r * 4 + 3) * CHUNK + g * L, L)] for r in range(3)]
                for j in range(STEP):
                    gj = t16 + cst(j * 12 * FP)
                    sj = row6 + cst(j)
                    ra = [
                        [
                            plsc.load_gather(tab_v, [gj + cst((k * 3 + c) * FP)])
                            for c in range(3)
                        ]
                        for k in range(3)
                    ]
                    ta = [
                        plsc.load_gather(tab_v, [gj + cst((9 + k) * FP)])
                        for k in range(3)
                    ]
                    for r in range(3):
                        for c in range(3):
                            acc = (
                                rb[r][0] * ra[0][c]
                                + rb[r][1] * ra[1][c]
                                + rb[r][2] * ra[2][c]
                            )
                            plsc.store_scatter(
                                r_v, [sj + cst((r * 3 + c) * C6)], acc
                            )
                        tv = (
                            rb[r][0] * ta[0]
                            + rb[r][1] * ta[1]
                            + rb[r][2] * ta[2]
                            + tb[r]
                        )
                        plsc.store_scatter(t_v, [sj + cst(r * C6)], tv)

        fire_in(0, 0)
        fire_in(1, 1)
        pltpu.sync_copy(der_hbm, tab_v)

        def pair_body(i, _):
            c0 = 2 * i
            for b in range(2):
                c = c0 + b
                wait_in(b)

                @pl.when(i > 0)
                def _():
                    wait_out(b)

                compute(b)
                fire_out(c, b)

                @pl.when(c + 2 < NCHUNK)
                def _():
                    fire_in(c + 2, b)

            return 0

        lax.fori_loop(0, NCHUNK // 2, pair_body, 0)
        wait_out(0)
        wait_out(1)

    return compose


def kernel(w2cs, time_index, blurry_res_rotation, blurry_res_transl):
    N = time_index.shape[0]
    F = blurry_res_rotation.shape[0]
    FP = ((F + 127) // 128) * 128
    derived = pl.pallas_call(
        _se3_table_body,
        out_shape=jax.ShapeDtypeStruct((72 * FP,), jnp.float32),
    )(blurry_res_rotation.T, blurry_res_transl.T)
    w_flat = jnp.transpose(w2cs, (1, 2, 0)).reshape(12 * N)
    tidx = time_index.astype(jnp.int32)
    r_flat, t_flat = _build_compose(N, F)(w_flat, tidx, derived)
    M = N * STEP
    r_new = r_flat.reshape(3, 3, M).transpose(2, 0, 1)
    t_new = t_flat.reshape(3, M).transpose(1, 0)
    return r_new, t_new


# speedup vs baseline: 1.0370x; 1.0370x over previous
import functools

import jax
import jax.numpy as jnp
from jax import lax
from jax.experimental import pallas as pl
from jax.experimental.pallas import tpu as pltpu
from jax.experimental.pallas import tpu_sc as plsc

EPS = float(jnp.finfo(jnp.float32).eps)
TRANSL_BIAS = 0.01
STEP = 6

NC, NS, L = 2, 16, 16
NW = NC * NS


def _se3_table_body(rot_ref, tra_ref, out_ref):
    F = rot_ref.shape[1]
    FP = out_ref.shape[0] // 72
    col = lax.broadcasted_iota(jnp.int32, (1, F), 1)
    mask = jnp.where((col >= 1) & (col < F - 1), 1.0, 0.0).astype(jnp.float32)
    for j in range(STEP):
        w0 = rot_ref[3 * j : 3 * j + 1, :] * mask + EPS
        w1 = rot_ref[3 * j + 1 : 3 * j + 2, :] * mask + EPS
        w2 = rot_ref[3 * j + 2 : 3 * j + 3, :] * mask + EPS
        u0 = tra_ref[3 * j : 3 * j + 1, :] * mask * TRANSL_BIAS
        u1 = tra_ref[3 * j + 1 : 3 * j + 2, :] * mask * TRANSL_BIAS
        u2 = tra_ref[3 * j + 2 : 3 * j + 3, :] * mask * TRANSL_BIAS
        sq0, sq1, sq2 = w0 * w0, w1 * w1, w2 * w2
        w01, w02, w12 = w0 * w1, w0 * w2, w1 * w2
        theta2 = sq0 + sq1 + sq2
        theta = jnp.sqrt(theta2)
        small = theta < 1e-6
        th = jnp.where(small, 1.0, theta)
        sin_t = jnp.sin(th)
        cos_t = jnp.cos(th)
        t2 = theta * theta
        A = jnp.where(small, 1.0 - t2 / 6.0, sin_t / th)
        B = jnp.where(small, 0.5 - t2 / 24.0, (1.0 - cos_t) / (th * th))
        C = jnp.where(small, 1.0 / 6.0 - t2 / 120.0, (th - sin_t) / (th * th * th))
        r00 = 1.0 - B * (sq1 + sq2)
        r01 = -A * w2 + B * w01
        r02 = A * w1 + B * w02
        r10 = A * w2 + B * w01
        r11 = 1.0 - B * (sq0 + sq2)
        r12 = -A * w0 + B * w12
        r20 = -A * w1 + B * w02
        r21 = A * w0 + B * w12
        r22 = 1.0 - B * (sq0 + sq1)
        v00 = 1.0 - C * (sq1 + sq2)
        v01 = -B * w2 + C * w01
        v02 = B * w1 + C * w02
        v10 = B * w2 + C * w01
        v11 = 1.0 - C * (sq0 + sq2)
        v12 = -B * w0 + C * w12
        v20 = -B * w1 + C * w02
        v21 = B * w0 + C * w12
        v22 = 1.0 - C * (sq0 + sq1)
        t0 = v00 * u0 + v01 * u1 + v02 * u2
        t1 = v10 * u0 + v11 * u1 + v12 * u2
        t2_ = v20 * u0 + v21 * u1 + v22 * u2
        rows = [r00, r01, r02, r10, r11, r12, r20, r21, r22, t0, t1, t2_]
        for d, v in enumerate(rows):
            out_ref[pl.ds((j * 12 + d) * FP, F)] = jnp.reshape(v, (F,))


@functools.lru_cache(maxsize=None)
def _build_compose(N, F):
    FP = ((F + 127) // 128) * 128
    RPW = N // NW
    CHUNK = 128
    NCHUNK = RPW // CHUNK
    GROUPS = CHUNK // L

    mesh = plsc.VectorSubcoreMesh(core_axis_name="c", subcore_axis_name="s")

    def cst(v):
        return jnp.full((L,), v, jnp.int32)

    @functools.partial(
        pl.kernel,
        mesh=mesh,
        out_type=[
            jax.ShapeDtypeStruct((9 * N * STEP,), jnp.float32),
            jax.ShapeDtypeStruct((3 * N * STEP,), jnp.float32),
        ],
        scratch_types=[
            pltpu.VMEM((72 * FP,), jnp.float32),
            [pltpu.VMEM((CHUNK,), jnp.int32) for _ in range(2)],
            [pltpu.VMEM((CHUNK * 12,), jnp.float32) for _ in range(2)],
            [pltpu.VMEM((CHUNK * 54,), jnp.float32) for _ in range(2)],
            [pltpu.VMEM((CHUNK * 18,), jnp.float32) for _ in range(2)],
            [pltpu.SemaphoreType.DMA for _ in range(2)],
            [pltpu.SemaphoreType.DMA for _ in range(2)],
        ],
        compiler_params=pltpu.CompilerParams(needs_layout_passes=False),
    )
    def compose(
        w_hbm, idx_hbm, der_hbm, r_hbm, t_hbm,
        tab_v, idx_vs, w_vs, r_vs, t_vs, in_sems, out_sems,
    ):
        wid = lax.axis_index("s") * NC + lax.axis_index("c")
        iota = lax.iota(jnp.int32, L)
        M = N * STEP
        C6 = CHUNK * 6

        def fire_in(c, b):
            base = wid * RPW + c * CHUNK
            pltpu.make_async_copy(
                idx_hbm.at[pl.ds(base, CHUNK)], idx_vs[b], in_sems[b]
            ).start()
            for d in range(12):
                pltpu.make_async_copy(
                    w_hbm.at[pl.ds(d * N + base, CHUNK)],
                    w_vs[b].at[pl.ds(d * CHUNK, CHUNK)],
                    in_sems[b],
                ).start()

        def wait_in(b):
            pltpu.make_async_copy(
                idx_hbm.at[pl.ds(0, CHUNK)], idx_vs[b], in_sems[b]
            ).wait()
            pltpu.make_async_copy(
                w_hbm.at[pl.ds(0, CHUNK * 12)], w_vs[b], in_sems[b]
            ).wait()

        def fire_out(c, b):
            base6 = (wid * RPW + c * CHUNK) * 6
            for d in range(9):
                pltpu.make_async_copy(
                    r_vs[b].at[pl.ds(d * C6, C6)],
                    r_hbm.at[pl.ds(d * M + base6, C6)],
                    out_sems[b],
                ).start()
            for d in range(3):
                pltpu.make_async_copy(
                    t_vs[b].at[pl.ds(d * C6, C6)],
                    t_hbm.at[pl.ds(d * M + base6, C6)],
                    out_sems[b],
                ).start()

        def wait_out(b):
            pltpu.make_async_copy(
                r_hbm.at[pl.ds(0, CHUNK * 54)], r_vs[b], out_sems[b]
            ).wait()
            pltpu.make_async_copy(
                t_hbm.at[pl.ds(0, CHUNK * 18)], t_vs[b], out_sems[b]
            ).wait()

        def compute(b):
            idx_v, w_v, r_v, t_v = idx_vs[b], w_vs[b], r_vs[b], t_vs[b]

            @plsc.parallel_loop(0, GROUPS, unroll=2)
            def group_body(g):
                row = g * L + iota
                t16 = idx_v[pl.ds(g * L, L)]
                row6 = row * 6
                rb = [
                    [w_v[pl.ds((r * 4 + k) * CHUNK + g * L, L)] for k in range(3)]
                    for r in range(3)
                ]
                tb = [w_v[pl.ds((r * 4 + 3) * CHUNK + g * L, L)] for r in range(3)]
                for j in range(STEP):
                    gj = t16 + cst(j * 12 * FP)
                    sj = row6 + cst(j)
                    ra = [
                        [
                            plsc.load_gather(tab_v, [gj + cst((k * 3 + c) * FP)])
                            for c in range(3)
                        ]
                        for k in range(3)
                    ]
                    ta = [
                        plsc.load_gather(tab_v, [gj + cst((9 + k) * FP)])
                        for k in range(3)
                    ]
                    for r in range(3):
                        for c in range(3):
                            acc = (
                                rb[r][0] * ra[0][c]
                                + rb[r][1] * ra[1][c]
                                + rb[r][2] * ra[2][c]
                            )
                            plsc.store_scatter(
                                r_v, [sj + cst((r * 3 + c) * C6)], acc
                            )
                        tv = (
                            rb[r][0] * ta[0]
                            + rb[r][1] * ta[1]
                            + rb[r][2] * ta[2]
                            + tb[r]
                        )
                        plsc.store_scatter(t_v, [sj + cst(r * C6)], tv)

        fire_in(0, 0)
        fire_in(1, 1)
        pltpu.sync_copy(der_hbm, tab_v)

        def pair_body(i, _):
            c0 = 2 * i
            for b in range(2):
                c = c0 + b
                wait_in(b)

                @pl.when(i > 0)
                def _():
                    wait_out(b)

                compute(b)
                fire_out(c, b)

                @pl.when(c + 2 < NCHUNK)
                def _():
                    fire_in(c + 2, b)

            return 0

        lax.fori_loop(0, NCHUNK // 2, pair_body, 0)
        wait_out(0)
        wait_out(1)

    return compose


def kernel(w2cs, time_index, blurry_res_rotation, blurry_res_transl):
    N = time_index.shape[0]
    F = blurry_res_rotation.shape[0]
    FP = ((F + 127) // 128) * 128
    derived = pl.pallas_call(
        _se3_table_body,
        out_shape=jax.ShapeDtypeStruct((72 * FP,), jnp.float32),
    )(blurry_res_rotation.T, blurry_res_transl.T)
    w_flat = jnp.transpose(w2cs, (1, 2, 0)).reshape(12 * N)
    tidx = time_index.astype(jnp.int32)
    r_flat, t_flat = _build_compose(N, F)(w_flat, tidx, derived)
    M = N * STEP
    r_new = r_flat.reshape(3, 3, M).transpose(2, 0, 1)
    t_new = t_flat.reshape(3, M).transpose(1, 0)
    return r_new, t_new


# speedup vs baseline: 1.0374x; 1.0004x over previous
import functools

import jax
import jax.numpy as jnp
from jax import lax
from jax.experimental import pallas as pl
from jax.experimental.pallas import tpu as pltpu
from jax.experimental.pallas import tpu_sc as plsc

EPS = float(jnp.finfo(jnp.float32).eps)
TRANSL_BIAS = 0.01
STEP = 6

NC, NS, L = 2, 16, 16
NW = NC * NS


def _se3_table_body(rot_ref, tra_ref, out_ref):
    F = rot_ref.shape[1]
    FP = out_ref.shape[0] // 72
    col = lax.broadcasted_iota(jnp.int32, (1, F), 1)
    mask = jnp.where((col >= 1) & (col < F - 1), 1.0, 0.0).astype(jnp.float32)
    for j in range(STEP):
        w0 = rot_ref[3 * j : 3 * j + 1, :] * mask + EPS
        w1 = rot_ref[3 * j + 1 : 3 * j + 2, :] * mask + EPS
        w2 = rot_ref[3 * j + 2 : 3 * j + 3, :] * mask + EPS
        u0 = tra_ref[3 * j : 3 * j + 1, :] * mask * TRANSL_BIAS
        u1 = tra_ref[3 * j + 1 : 3 * j + 2, :] * mask * TRANSL_BIAS
        u2 = tra_ref[3 * j + 2 : 3 * j + 3, :] * mask * TRANSL_BIAS
        sq0, sq1, sq2 = w0 * w0, w1 * w1, w2 * w2
        w01, w02, w12 = w0 * w1, w0 * w2, w1 * w2
        theta2 = sq0 + sq1 + sq2
        theta = jnp.sqrt(theta2)
        small = theta < 1e-6
        th = jnp.where(small, 1.0, theta)
        sin_t = jnp.sin(th)
        cos_t = jnp.cos(th)
        t2 = theta * theta
        A = jnp.where(small, 1.0 - t2 / 6.0, sin_t / th)
        B = jnp.where(small, 0.5 - t2 / 24.0, (1.0 - cos_t) / (th * th))
        C = jnp.where(small, 1.0 / 6.0 - t2 / 120.0, (th - sin_t) / (th * th * th))
        r00 = 1.0 - B * (sq1 + sq2)
        r01 = -A * w2 + B * w01
        r02 = A * w1 + B * w02
        r10 = A * w2 + B * w01
        r11 = 1.0 - B * (sq0 + sq2)
        r12 = -A * w0 + B * w12
        r20 = -A * w1 + B * w02
        r21 = A * w0 + B * w12
        r22 = 1.0 - B * (sq0 + sq1)
        v00 = 1.0 - C * (sq1 + sq2)
        v01 = -B * w2 + C * w01
        v02 = B * w1 + C * w02
        v10 = B * w2 + C * w01
        v11 = 1.0 - C * (sq0 + sq2)
        v12 = -B * w0 + C * w12
        v20 = -B * w1 + C * w02
        v21 = B * w0 + C * w12
        v22 = 1.0 - C * (sq0 + sq1)
        t0 = v00 * u0 + v01 * u1 + v02 * u2
        t1 = v10 * u0 + v11 * u1 + v12 * u2
        t2_ = v20 * u0 + v21 * u1 + v22 * u2
        rows = [r00, r01, r02, r10, r11, r12, r20, r21, r22, t0, t1, t2_]
        for d, v in enumerate(rows):
            out_ref[pl.ds((j * 12 + d) * FP, F)] = jnp.reshape(v, (F,))


@functools.lru_cache(maxsize=None)
def _build_compose(N, F):
    FP = ((F + 127) // 128) * 128
    RPW = N // NW
    CHUNK = 128
    NCHUNK = RPW // CHUNK
    GROUPS = CHUNK // L

    mesh = plsc.VectorSubcoreMesh(core_axis_name="c", subcore_axis_name="s")

    def cst(v):
        return jnp.full((L,), v, jnp.int32)

    @functools.partial(
        pl.kernel,
        mesh=mesh,
        out_type=[
            jax.ShapeDtypeStruct((9 * N * STEP,), jnp.float32),
            jax.ShapeDtypeStruct((3 * N * STEP,), jnp.float32),
        ],
        scratch_types=[
            pltpu.VMEM((72 * FP,), jnp.float32),
            [pltpu.VMEM((CHUNK,), jnp.int32) for _ in range(2)],
            [pltpu.VMEM((CHUNK * 12,), jnp.float32) for _ in range(2)],
            [pltpu.VMEM((CHUNK * 54,), jnp.float32) for _ in range(2)],
            [pltpu.VMEM((CHUNK * 18,), jnp.float32) for _ in range(2)],
            [pltpu.SemaphoreType.DMA for _ in range(2)],
            [pltpu.SemaphoreType.DMA for _ in range(2)],
            pltpu.SemaphoreType.DMA,
        ],
        compiler_params=pltpu.CompilerParams(needs_layout_passes=False),
    )
    def compose(
        w_hbm, idx_hbm, der_hbm, r_hbm, t_hbm,
        tab_v, idx_vs, w_vs, r_vs, t_vs, in_sems, out_sems, tab_sem,
    ):
        wid = lax.axis_index("s") * NC + lax.axis_index("c")
        iota = lax.iota(jnp.int32, L)
        M = N * STEP
        C6 = CHUNK * 6

        def fire_in(c, b):
            base = wid * RPW + c * CHUNK
            pltpu.make_async_copy(
                idx_hbm.at[pl.ds(base, CHUNK)], idx_vs[b], in_sems[b]
            ).start()
            for d in range(12):
                pltpu.make_async_copy(
                    w_hbm.at[pl.ds(d * N + base, CHUNK)],
                    w_vs[b].at[pl.ds(d * CHUNK, CHUNK)],
                    in_sems[b],
                ).start()

        def wait_in(b):
            pltpu.make_async_copy(
                idx_hbm.at[pl.ds(0, CHUNK)], idx_vs[b], in_sems[b]
            ).wait()
            pltpu.make_async_copy(
                w_hbm.at[pl.ds(0, CHUNK * 12)], w_vs[b], in_sems[b]
            ).wait()

        def fire_out(c, b):
            base6 = (wid * RPW + c * CHUNK) * 6
            for d in range(9):
                pltpu.make_async_copy(
                    r_vs[b].at[pl.ds(d * C6, C6)],
                    r_hbm.at[pl.ds(d * M + base6, C6)],
                    out_sems[b],
                ).start()
            for d in range(3):
                pltpu.make_async_copy(
                    t_vs[b].at[pl.ds(d * C6, C6)],
                    t_hbm.at[pl.ds(d * M + base6, C6)],
                    out_sems[b],
                ).start()

        def wait_out(b):
            pltpu.make_async_copy(
                r_hbm.at[pl.ds(0, CHUNK * 54)], r_vs[b], out_sems[b]
            ).wait()
            pltpu.make_async_copy(
                t_hbm.at[pl.ds(0, CHUNK * 18)], t_vs[b], out_sems[b]
            ).wait()

        def compute(b):
            idx_v, w_v, r_v, t_v = idx_vs[b], w_vs[b], r_vs[b], t_vs[b]

            @plsc.parallel_loop(0, GROUPS, unroll=2)
            def group_body(g):
                row = g * L + iota
                t16 = idx_v[pl.ds(g * L, L)]
                row6 = row * 6
                rb = [
                    [w_v[pl.ds((r * 4 + k) * CHUNK + g * L, L)] for k in range(3)]
                    for r in range(3)
                ]
                tb = [w_v[pl.ds((r * 4 + 3) * CHUNK + g * L, L)] for r in range(3)]
                for j in range(STEP):
                    gj = t16 + cst(j * 12 * FP)
                    sj = row6 + cst(j)
                    ra = [
                        [
                            plsc.load_gather(tab_v, [gj + cst((k * 3 + c) * FP)])
                            for c in range(3)
                        ]
                        for k in range(3)
                    ]
                    ta = [
                        plsc.load_gather(tab_v, [gj + cst((9 + k) * FP)])
                        for k in range(3)
                    ]
                    for r in range(3):
                        for c in range(3):
                            acc = (
                                rb[r][0] * ra[0][c]
                                + rb[r][1] * ra[1][c]
                                + rb[r][2] * ra[2][c]
                            )
                            plsc.store_scatter(
                                r_v, [sj + cst((r * 3 + c) * C6)], acc
                            )
                        tv = (
                            rb[r][0] * ta[0]
                            + rb[r][1] * ta[1]
                            + rb[r][2] * ta[2]
                            + tb[r]
                        )
                        plsc.store_scatter(t_v, [sj + cst(r * C6)], tv)

        tab_copy = pltpu.make_async_copy(der_hbm, tab_v, tab_sem)
        tab_copy.start()
        fire_in(0, 0)
        fire_in(1, 1)
        tab_copy.wait()

        def pair_body(i, _):
            c0 = 2 * i
            for b in range(2):
                c = c0 + b
                wait_in(b)

                @pl.when(i > 0)
                def _():
                    wait_out(b)

                compute(b)
                fire_out(c, b)

                @pl.when(c + 2 < NCHUNK)
                def _():
                    fire_in(c + 2, b)

            return 0

        lax.fori_loop(0, NCHUNK // 2, pair_body, 0)
        wait_out(0)
        wait_out(1)

    return compose


def kernel(w2cs, time_index, blurry_res_rotation, blurry_res_transl):
    N = time_index.shape[0]
    F = blurry_res_rotation.shape[0]
    FP = ((F + 127) // 128) * 128
    derived = pl.pallas_call(
        _se3_table_body,
        out_shape=jax.ShapeDtypeStruct((72 * FP,), jnp.float32),
    )(blurry_res_rotation.T, blurry_res_transl.T)
    w_flat = jnp.transpose(w2cs, (1, 2, 0)).reshape(12 * N)
    tidx = time_index.astype(jnp.int32)
    r_flat, t_flat = _build_compose(N, F)(w_flat, tidx, derived)
    M = N * STEP
    r_new = r_flat.reshape(3, 3, M).transpose(2, 0, 1)
    t_new = t_flat.reshape(3, M).transpose(1, 0)
    return r_new, t_new
